# baseline (speedup 1.0000x reference)
import functools

import jax
import jax.numpy as jnp
from jax import lax
from jax.experimental import pallas as pl
from jax.experimental.pallas import tpu as pltpu
from jax.experimental.pallas import tpu_sc as plsc


def _make_gather(V, D, B):
    info = plsc.get_sparse_core_info()
    NC, NS = info.num_cores, info.num_subcores
    NW = NC * NS
    assert B % (8 * NW) == 0
    b_per_w = B // NW
    mesh = plsc.VectorSubcoreMesh(core_axis_name="c", subcore_axis_name="s")

    @functools.partial(
        pl.kernel,
        mesh=mesh,
        out_type=jax.ShapeDtypeStruct((B, D), jnp.float32),
        scratch_types=[
            pltpu.VMEM((b_per_w,), jnp.int32),
            pltpu.VMEM((b_per_w, D), jnp.float32),
            pltpu.SemaphoreType.DMA,
        ],
        compiler_params=pltpu.CompilerParams(use_tc_tiling_on_sc=False),
    )
    def k(table_hbm, idx_hbm, out_hbm, idx_v, rows_v, sem):
        wid = lax.axis_index("s") * NC + lax.axis_index("c")
        base = wid * b_per_w
        pltpu.sync_copy(idx_hbm.at[pl.ds(base, b_per_w)], idx_v)
        pltpu.async_copy(table_hbm.at[idx_v], rows_v, sem).wait()
        pltpu.sync_copy(rows_v, out_hbm.at[pl.ds(base, b_per_w)])

    return k


def kernel(batch, table):
    B = batch.shape[0]
    V, D = table.shape
    return _make_gather(V, D, B)(table, batch.astype(jnp.int32))


# speedup vs baseline: 5.9044x; 5.9044x over previous
import functools

import jax
import jax.numpy as jnp
from jax import lax
from jax.experimental import pallas as pl
from jax.experimental.pallas import tpu as pltpu
from jax.experimental.pallas import tpu_sc as plsc

_LANES = 16
_BATCH = 16


def _make_gather(V, D, B):
    info = plsc.get_sparse_core_info()
    NC, NS = info.num_cores, info.num_subcores
    NW = NC * NS
    assert B % (8 * NW) == 0
    b_per_w = B // NW
    nbatches = b_per_w // _BATCH
    mesh = plsc.VectorSubcoreMesh(core_axis_name="c", subcore_axis_name="s")

    @functools.partial(
        pl.kernel,
        mesh=mesh,
        out_type=jax.ShapeDtypeStruct((D, B), jnp.float32),
        scratch_types=[
            pltpu.VMEM((b_per_w,), jnp.int32),
            pltpu.VMEM((2 * _BATCH, D, 128), jnp.float32),
            pltpu.VMEM((D, b_per_w), jnp.float32),
            pltpu.SemaphoreType.DMA,
            pltpu.SemaphoreType.DMA,
        ],
        compiler_params=pltpu.CompilerParams(needs_layout_passes=False),
    )
    def k(tableT_hbm, idx_hbm, outT_hbm, idx_v, ring_v, buf_v, sem_a, sem_b):
        wid = lax.axis_index("s") * NC + lax.axis_index("c")
        base = wid * b_per_w
        pltpu.sync_copy(idx_hbm.at[pl.ds(base, b_per_w)], idx_v)
        jcol = lax.iota(jnp.int32, _LANES)

        def issue(g, sem):
            rv = idx_v[pl.ds(g * _BATCH, _BATCH)]
            for t in range(_BATCH):
                c = (rv[t] >> 7) * 128
                slot = (g % 2) * _BATCH + t
                pltpu.async_copy(
                    tableT_hbm.at[:, pl.ds(c, 128)], ring_v.at[slot], sem
                )

        def drain_extract(g, sem):
            for t in range(_BATCH):
                slot = (g % 2) * _BATCH + t
                pltpu.make_async_copy(
                    tableT_hbm.at[:, pl.ds(0, 128)], ring_v.at[slot], sem
                ).wait()
            rv = idx_v[pl.ds(g * _BATCH, _BATCH)]
            colv = rv & 127
            for t in range(_BATCH):
                i = g * _BATCH + t
                slot = (g % 2) * _BATCH + t
                vals = plsc.load_gather(
                    ring_v,
                    [
                        jnp.full((_LANES,), slot, jnp.int32),
                        jcol,
                        jnp.full((_LANES,), colv[t], jnp.int32),
                    ],
                )
                plsc.store_scatter(
                    buf_v, [jcol, jnp.full((_LANES,), i, jnp.int32)], vals
                )

        issue(0, sem_a)
        issue(1, sem_b)

        def body(g, _):
            drain_extract(g, sem_a)

            @pl.when(g + 2 < nbatches)
            def _():
                issue(g + 2, sem_a)

            drain_extract(g + 1, sem_b)

            @pl.when(g + 3 < nbatches)
            def _():
                issue(g + 3, sem_b)

            return 0

        lax.fori_loop(0, nbatches // 2, lambda p, c: body(p * 2, c), 0)
        pltpu.sync_copy(buf_v, outT_hbm.at[:, pl.ds(base, b_per_w)])

    return k


def kernel(batch, table):
    B = batch.shape[0]
    V, D = table.shape
    outT = _make_gather(V, D, B)(table.T, batch.astype(jnp.int32))
    return outT.T
